# baseline (speedup 1.0000x reference)
import functools

import jax
import jax.numpy as jnp
from jax import lax
from jax.experimental import pallas as pl
from jax.experimental.pallas import tpu as pltpu
from jax.experimental.pallas import tpu_sc as plsc

NC = 2
NS = 16
NW = NC * NS
L = 16
NACC = 5


def _build(n_pix, chunk_pix, interpret=False):
    assert n_pix % NW == 0
    pix_w = n_pix // NW
    assert pix_w % chunk_pix == 0
    n_chunk = pix_w // chunk_pix
    groups = chunk_pix // L
    mesh = plsc.VectorSubcoreMesh(core_axis_name="c", subcore_axis_name="s",
                                  num_cores=NC, num_subcores=NS)

    @functools.partial(
        pl.kernel,
        out_type=jax.ShapeDtypeStruct((NW * NACC * L,), jnp.float32),
        mesh=mesh,
        interpret=interpret,
        compiler_params=pltpu.CompilerParams(needs_layout_passes=False),
        scratch_types=[
            pltpu.VMEM((chunk_pix * 3,), jnp.float32),
            pltpu.VMEM((chunk_pix * 3,), jnp.float32),
            pltpu.VMEM((chunk_pix,), jnp.int32),
            pltpu.VMEM((chunk_pix,), jnp.int32),
            pltpu.VMEM((NACC * L,), jnp.float32),
            pltpu.SemaphoreType.DMA,
            pltpu.SemaphoreType.DMA,
            pltpu.SemaphoreType.DMA,
            pltpu.SemaphoreType.DMA,
        ],
    )
    def phase1(feats_hbm, inst_hbm, part_hbm, fbuf0, fbuf1, ibuf0, ibuf1,
               obuf, semf0, semf1, semi0, semi1):
        wid = lax.axis_index("s") * NC + lax.axis_index("c")
        base_pix = wid * pix_w
        fbuf = [fbuf0, fbuf1]
        ibuf = [ibuf0, ibuf1]
        semf = [semf0, semf1]
        semi = [semi0, semi1]
        iota = lax.iota(jnp.int32, L)
        iota3 = iota * 3

        def start(g, slot):
            b = base_pix + g * chunk_pix
            hf = pltpu.async_copy(
                feats_hbm.at[pl.ds(b * 3, chunk_pix * 3)], fbuf[slot],
                semf[slot])
            hi = pltpu.async_copy(
                inst_hbm.at[pl.ds(b, chunk_pix)], ibuf[slot], semi[slot])
            return hf, hi

        pend = [None, None]
        pend[0] = start(0, 0)
        zeros = jnp.zeros((L,), jnp.float32)
        carry = (zeros, zeros, zeros, zeros, zeros)
        for g in range(n_chunk):
            slot = g % 2
            if g + 1 < n_chunk:
                pend[(g + 1) % 2] = start(g + 1, (g + 1) % 2)
            hf, hi = pend[slot]
            hf.wait()
            hi.wait()
            fslot = fbuf[slot]
            islot = ibuf[slot]

            def body(i, acc, fslot=fslot, islot=islot):
                t0, t1, t2, c1, c2 = acc
                c = islot[pl.ds(i * L, L)].astype(jnp.float32)
                idx0 = iota3 + i * (3 * L)
                f0 = plsc.load_gather(fslot, [idx0])
                f1 = plsc.load_gather(fslot, [idx0 + 1])
                f2 = plsc.load_gather(fslot, [idx0 + 2])
                t = f0 + f1 + f2
                x = t * c
                return (t0 + t, t1 + x, t2 + x * c, c1 + c, c2 + c * c)

            carry = lax.fori_loop(0, groups, body, carry)

        for a in range(NACC):
            obuf[pl.ds(a * L, L)] = carry[a]
        pltpu.sync_copy(obuf, part_hbm.at[pl.ds(wid * NACC * L, NACC * L)])

    @functools.partial(
        pl.kernel,
        out_type=jax.ShapeDtypeStruct((n_pix * 3,), jnp.float32),
        mesh=mesh,
        interpret=interpret,
        compiler_params=pltpu.CompilerParams(needs_layout_passes=False),
        scratch_types=[
            pltpu.VMEM((chunk_pix,), jnp.int32),
            pltpu.VMEM((chunk_pix,), jnp.int32),
            pltpu.VMEM((chunk_pix * 3,), jnp.float32),
            pltpu.VMEM((chunk_pix * 3,), jnp.float32),
            pltpu.VMEM((NW * NACC * L,), jnp.float32),
            pltpu.SemaphoreType.DMA,
            pltpu.SemaphoreType.DMA,
            pltpu.SemaphoreType.DMA,
            pltpu.SemaphoreType.DMA,
        ],
    )
    def phase2(inst_hbm, part_hbm, out_hbm, ibuf0, ibuf1, obuf0, obuf1,
               pbuf, semi0, semi1, semo0, semo1):
        wid = lax.axis_index("s") * NC + lax.axis_index("c")
        base_pix = wid * pix_w
        ibuf = [ibuf0, ibuf1]
        obuf = [obuf0, obuf1]
        semi = [semi0, semi1]
        semo = [semo0, semo1]
        iota = lax.iota(jnp.int32, L)
        iota3 = iota * 3

        pltpu.sync_copy(part_hbm, pbuf)
        accs = []
        for a in range(NACC):
            v = jnp.zeros((L,), jnp.float32)
            for w in range(NW):
                v = v + pbuf[pl.ds((w * NACC + a) * L, L)]
            accs.append(jnp.sum(v))
        t0, t1, t2, c1, c2 = accs
        n_elems = jnp.float32(n_pix * 3)
        s2 = (t2 - t1) * 0.5
        s1 = t1 - 2.0 * s2
        s0 = t0 - s1 - s2
        n2 = 3.0 * (c2 - c1) * 0.5
        n1 = 3.0 * c1 - 2.0 * n2
        n0 = n_elems - n1 - n2
        m0 = jnp.full((L,), s0, jnp.float32) / jnp.full((L,), n0, jnp.float32)
        m1 = jnp.full((L,), s1, jnp.float32) / jnp.full((L,), n1, jnp.float32)
        m2 = jnp.full((L,), s2, jnp.float32) / jnp.full((L,), n2, jnp.float32)

        def start_in(g, slot):
            b = base_pix + g * chunk_pix
            return pltpu.async_copy(
                inst_hbm.at[pl.ds(b, chunk_pix)], ibuf[slot], semi[slot])

        pend_in = [None, None]
        pend_out = [None, None]
        pend_in[0] = start_in(0, 0)
        for g in range(n_chunk):
            slot = g % 2
            if g + 1 < n_chunk:
                pend_in[(g + 1) % 2] = start_in(g + 1, (g + 1) % 2)
            pend_in[slot].wait()
            if pend_out[slot] is not None:
                pend_out[slot].wait()
            islot = ibuf[slot]
            oslot = obuf[slot]

            def body(i, _, islot=islot, oslot=oslot):
                ci = islot[pl.ds(i * L, L)]
                v = jnp.where(ci == 0, m0, jnp.where(ci == 1, m1, m2))
                idx0 = iota3 + i * (3 * L)
                plsc.store_scatter(oslot, [idx0], v)
                plsc.store_scatter(oslot, [idx0 + 1], v)
                plsc.store_scatter(oslot, [idx0 + 2], v)
                return 0

            lax.fori_loop(0, groups, body, 0)
            b = base_pix + g * chunk_pix
            pend_out[slot] = pltpu.async_copy(
                oslot, out_hbm.at[pl.ds(b * 3, chunk_pix * 3)], semo[slot])
        for slot in range(2):
            if pend_out[slot] is not None:
                pend_out[slot].wait()

    def run(feats_flat, inst_flat):
        part = phase1(feats_flat, inst_flat)
        return phase2(inst_flat, part)

    return run


_build_cached = functools.lru_cache(maxsize=None)(_build)


@jax.jit
def kernel(feats, inst):
    n_pix = feats.shape[0] * feats.shape[1] * feats.shape[2]
    run = _build_cached(n_pix, 8192)
    ff = feats.reshape(-1)
    ii = inst.reshape(-1)
    out = run(ff, ii)
    return out.reshape(feats.shape)


# speedup vs baseline: 55.2905x; 55.2905x over previous
import functools

import jax
import jax.numpy as jnp
from jax import lax
from jax.experimental import pallas as pl
from jax.experimental.pallas import tpu as pltpu
from jax.experimental.pallas import tpu_sc as plsc

NC = 2
NS = 16
NW = NC * NS
L = 16
NACC = 5

B, H, W, C = 8, 512, 512, 3
N_PIX = B * H * W
PIX_W = N_PIX // NW
HROWS_W = PIX_W // W
N_CHUNK = 8
CH_H = HROWS_W // N_CHUNK
CHUNK_PIX = CH_H * W
GROUPS = CHUNK_PIX // L
IROWS = CHUNK_PIX // 128

_params = pltpu.CompilerParams(use_tc_tiling_on_sc=True,
                               needs_layout_passes=False)


def _wid():
    return lax.axis_index("s") * NC + lax.axis_index("c")


def _mesh():
    return plsc.VectorSubcoreMesh(core_axis_name="c", subcore_axis_name="s",
                                  num_cores=NC, num_subcores=NS)


def _make_kernels():
    mesh = _mesh()

    @functools.partial(
        pl.kernel,
        out_type=jax.ShapeDtypeStruct((NW, 8, 128), jnp.float32),
        mesh=mesh,
        compiler_params=_params,
        scratch_types=[
            pltpu.VMEM((3 * CH_H, W), jnp.float32),
            pltpu.VMEM((3 * CH_H, W), jnp.float32),
            pltpu.VMEM((IROWS, 128), jnp.int32),
            pltpu.VMEM((IROWS, 128), jnp.int32),
            pltpu.VMEM((8, 128), jnp.float32),
            pltpu.SemaphoreType.DMA,
            pltpu.SemaphoreType.DMA,
            pltpu.SemaphoreType.DMA,
            pltpu.SemaphoreType.DMA,
        ],
    )
    def phase1(feats_hbm, inst_hbm, part_hbm, fbuf0, fbuf1, ibuf0, ibuf1,
               obuf, semf0, semf1, semi0, semi1):
        wid = _wid()
        b = wid // 4
        q = wid % 4
        h0 = q * HROWS_W
        fbase = b * (C * H) + h0
        irow0 = wid * (PIX_W // 128)
        fbuf = [fbuf0, fbuf1]
        ibuf = [ibuf0, ibuf1]
        semf = [semf0, semf1]
        semi = [semi0, semi1]

        def start(g, slot):
            hs = []
            for c in range(C):
                r = fbase + c * H + g * CH_H
                hs.append(pltpu.async_copy(
                    feats_hbm.at[pl.ds(r, CH_H), :],
                    fbuf[slot].at[pl.ds(c * CH_H, CH_H), :], semf[slot]))
            hs.append(pltpu.async_copy(
                inst_hbm.at[pl.ds(irow0 + g * IROWS, IROWS), :],
                ibuf[slot], semi[slot]))
            return hs

        pend = [None, None]
        pend[0] = start(0, 0)
        zeros = jnp.zeros((L,), jnp.float32)
        carry = (zeros, zeros, zeros, zeros, zeros)
        for g in range(N_CHUNK):
            slot = g % 2
            if g + 1 < N_CHUNK:
                pend[(g + 1) % 2] = start(g + 1, (g + 1) % 2)
            for hdl in pend[slot]:
                hdl.wait()
            fslot = fbuf[slot]
            islot = ibuf[slot]

            def body(i, acc, fslot=fslot, islot=islot):
                t0, t1, t2, c1, c2 = acc
                hr = i >> 5
                wc = pl.multiple_of((i & 31) << 4, 16)
                ir = i >> 3
                ic = pl.multiple_of((i & 7) << 4, 16)
                cv = islot[ir, pl.ds(ic, L)].astype(jnp.float32)
                t = (fslot[hr, pl.ds(wc, L)]
                     + fslot[hr + CH_H, pl.ds(wc, L)]
                     + fslot[hr + 2 * CH_H, pl.ds(wc, L)])
                x = t * cv
                return (t0 + t, t1 + x, t2 + x * cv, c1 + cv, c2 + cv * cv)

            carry = lax.fori_loop(0, GROUPS, body, carry, unroll=4)

        for a in range(NACC):
            obuf[0, pl.ds(a * L, L)] = carry[a]
        pltpu.sync_copy(obuf, part_hbm.at[wid])

    @functools.partial(
        pl.kernel,
        out_type=jax.ShapeDtypeStruct((B * C * H, W), jnp.float32),
        mesh=mesh,
        compiler_params=_params,
        scratch_types=[
            pltpu.VMEM((IROWS, 128), jnp.int32),
            pltpu.VMEM((IROWS, 128), jnp.int32),
            pltpu.VMEM((3 * CH_H, W), jnp.float32),
            pltpu.VMEM((3 * CH_H, W), jnp.float32),
            pltpu.VMEM((NW, 8, 128), jnp.float32),
            pltpu.SemaphoreType.DMA,
            pltpu.SemaphoreType.DMA,
            pltpu.SemaphoreType.DMA,
            pltpu.SemaphoreType.DMA,
        ],
    )
    def phase2(inst_hbm, part_hbm, out_hbm, ibuf0, ibuf1, obuf0, obuf1,
               pbuf, semi0, semi1, semo0, semo1):
        wid = _wid()
        b = wid // 4
        q = wid % 4
        h0 = q * HROWS_W
        fbase = b * (C * H) + h0
        irow0 = wid * (PIX_W // 128)
        ibuf = [ibuf0, ibuf1]
        obuf = [obuf0, obuf1]
        semi = [semi0, semi1]
        semo = [semo0, semo1]

        pltpu.sync_copy(part_hbm, pbuf)
        accs = []
        for a in range(NACC):
            v = jnp.zeros((L,), jnp.float32)
            for w in range(NW):
                v = v + pbuf[w, 0, pl.ds(a * L, L)]
            accs.append(jnp.sum(v))
        t0, t1, t2, c1, c2 = accs
        n_elems = jnp.float32(N_PIX * 3)
        s2 = (t2 - t1) * 0.5
        s1 = t1 - 2.0 * s2
        s0 = t0 - s1 - s2
        n2 = 3.0 * (c2 - c1) * 0.5
        n1 = 3.0 * c1 - 2.0 * n2
        n0 = n_elems - n1 - n2
        m0 = jnp.full((L,), s0, jnp.float32) / jnp.full((L,), n0, jnp.float32)
        m1 = jnp.full((L,), s1, jnp.float32) / jnp.full((L,), n1, jnp.float32)
        m2 = jnp.full((L,), s2, jnp.float32) / jnp.full((L,), n2, jnp.float32)

        def start_in(g, slot):
            return pltpu.async_copy(
                inst_hbm.at[pl.ds(irow0 + g * IROWS, IROWS), :],
                ibuf[slot], semi[slot])

        pend_in = [None, None]
        pend_out = [None, None]
        pend_in[0] = start_in(0, 0)
        for g in range(N_CHUNK):
            slot = g % 2
            if g + 1 < N_CHUNK:
                pend_in[(g + 1) % 2] = start_in(g + 1, (g + 1) % 2)
            pend_in[slot].wait()
            if pend_out[slot] is not None:
                for hdl in pend_out[slot]:
                    hdl.wait()
            islot = ibuf[slot]
            oslot = obuf[slot]

            def body(i, _, islot=islot, oslot=oslot):
                hr = i >> 5
                wc = pl.multiple_of((i & 31) << 4, 16)
                ir = i >> 3
                ic = pl.multiple_of((i & 7) << 4, 16)
                ci = islot[ir, pl.ds(ic, L)]
                v = jnp.where(ci == 0, m0, jnp.where(ci == 1, m1, m2))
                oslot[hr, pl.ds(wc, L)] = v
                oslot[hr + CH_H, pl.ds(wc, L)] = v
                oslot[hr + 2 * CH_H, pl.ds(wc, L)] = v
                return 0

            lax.fori_loop(0, GROUPS, body, 0, unroll=4)
            hs = []
            for c in range(C):
                r = fbase + c * H + g * CH_H
                hs.append(pltpu.async_copy(
                    oslot.at[pl.ds(c * CH_H, CH_H), :],
                    out_hbm.at[pl.ds(r, CH_H), :], semo[slot]))
            pend_out[slot] = hs
        for slot in range(2):
            if pend_out[slot] is not None:
                for hdl in pend_out[slot]:
                    hdl.wait()

    def run(feats, inst):
        ft = feats.transpose(0, 3, 1, 2).reshape(B * C * H, W)
        ii = inst.reshape(N_PIX // 128, 128)
        part = phase1(ft, ii)
        out2d = phase2(ii, part)
        return out2d.reshape(B, C, H, W).transpose(0, 2, 3, 1)

    return run


_make_kernels_cached = functools.lru_cache(maxsize=None)(_make_kernels)


@jax.jit
def kernel(feats, inst):
    return _make_kernels_cached()(feats, inst)


# speedup vs baseline: 58.7096x; 1.0618x over previous
import functools

import jax
import jax.numpy as jnp
from jax import lax
from jax.experimental import pallas as pl
from jax.experimental.pallas import tpu as pltpu
from jax.experimental.pallas import tpu_sc as plsc

NC = 2
NS = 16
NW = NC * NS
L = 16
NACC = 5

B, H, W, C = 8, 512, 512, 3
N_PIX = B * H * W
PIX_W = N_PIX // NW
HROWS_W = PIX_W // W
N_CHUNK = 8
CH_H = HROWS_W // N_CHUNK
CHUNK_PIX = CH_H * W
GROUPS = CHUNK_PIX // L
IROWS = CHUNK_PIX // 128

_params = pltpu.CompilerParams(use_tc_tiling_on_sc=True,
                               needs_layout_passes=False)


def _wid():
    return lax.axis_index("s") * NC + lax.axis_index("c")


def _mesh():
    return plsc.VectorSubcoreMesh(core_axis_name="c", subcore_axis_name="s",
                                  num_cores=NC, num_subcores=NS)


def _make_kernels():
    mesh = _mesh()

    @functools.partial(
        pl.kernel,
        out_type=jax.ShapeDtypeStruct((NW, 8, 128), jnp.float32),
        mesh=mesh,
        compiler_params=_params,
        scratch_types=[
            pltpu.VMEM((3 * CH_H, W), jnp.float32),
            pltpu.VMEM((3 * CH_H, W), jnp.float32),
            pltpu.VMEM((IROWS, 128), jnp.int32),
            pltpu.VMEM((IROWS, 128), jnp.int32),
            pltpu.VMEM((8, 128), jnp.float32),
            pltpu.SemaphoreType.DMA,
            pltpu.SemaphoreType.DMA,
            pltpu.SemaphoreType.DMA,
            pltpu.SemaphoreType.DMA,
        ],
    )
    def phase1(feats_hbm, inst_hbm, part_hbm, fbuf0, fbuf1, ibuf0, ibuf1,
               obuf, semf0, semf1, semi0, semi1):
        wid = _wid()
        b = wid // 4
        q = wid % 4
        h0 = q * HROWS_W
        fbase = b * (C * H) + h0
        irow0 = wid * (PIX_W // 128)
        fbuf = [fbuf0, fbuf1]
        ibuf = [ibuf0, ibuf1]
        semf = [semf0, semf1]
        semi = [semi0, semi1]

        def start(g, slot):
            hs = []
            for c in range(C):
                r = fbase + c * H + g * CH_H
                hs.append(pltpu.async_copy(
                    feats_hbm.at[pl.ds(r, CH_H), :],
                    fbuf[slot].at[pl.ds(c * CH_H, CH_H), :], semf[slot]))
            hs.append(pltpu.async_copy(
                inst_hbm.at[pl.ds(irow0 + g * IROWS, IROWS), :],
                ibuf[slot], semi[slot]))
            return hs

        pend = [None, None]
        pend[0] = start(0, 0)
        zeros = jnp.zeros((L,), jnp.float32)
        carry = (zeros, zeros, zeros, zeros, zeros)
        for g in range(N_CHUNK):
            slot = g % 2
            if g + 1 < N_CHUNK:
                pend[(g + 1) % 2] = start(g + 1, (g + 1) % 2)
            for hdl in pend[slot]:
                hdl.wait()
            fslot = fbuf[slot]
            islot = ibuf[slot]

            def body(i, acc, fslot=fslot, islot=islot):
                t0, t1, t2, c1, c2 = acc
                hr = i >> 5
                wc = pl.multiple_of((i & 31) << 4, 16)
                ir = i >> 3
                ic = pl.multiple_of((i & 7) << 4, 16)
                cv = islot[ir, pl.ds(ic, L)].astype(jnp.float32)
                t = (fslot[hr, pl.ds(wc, L)]
                     + fslot[hr + CH_H, pl.ds(wc, L)]
                     + fslot[hr + 2 * CH_H, pl.ds(wc, L)])
                x = t * cv
                return (t0 + t, t1 + x, t2 + x * cv, c1 + cv, c2 + cv * cv)

            carry = lax.fori_loop(0, GROUPS, body, carry, unroll=4)

        for a in range(NACC):
            obuf[0, pl.ds(a * L, L)] = carry[a]
        pltpu.sync_copy(obuf, part_hbm.at[wid])

    @functools.partial(
        pl.kernel,
        out_type=jax.ShapeDtypeStruct((B * C * H, W), jnp.float32),
        mesh=mesh,
        compiler_params=_params,
        scratch_types=[
            pltpu.VMEM((IROWS, 128), jnp.int32),
            pltpu.VMEM((IROWS, 128), jnp.int32),
            pltpu.VMEM((CH_H, W), jnp.float32),
            pltpu.VMEM((CH_H, W), jnp.float32),
            pltpu.VMEM((NW, 8, 128), jnp.float32),
            pltpu.SemaphoreType.DMA,
            pltpu.SemaphoreType.DMA,
            pltpu.SemaphoreType.DMA,
            pltpu.SemaphoreType.DMA,
        ],
    )
    def phase2(inst_hbm, part_hbm, out_hbm, ibuf0, ibuf1, obuf0, obuf1,
               pbuf, semi0, semi1, semo0, semo1):
        wid = _wid()
        b = wid // 4
        q = wid % 4
        h0 = q * HROWS_W
        fbase = b * (C * H) + h0
        irow0 = wid * (PIX_W // 128)
        ibuf = [ibuf0, ibuf1]
        obuf = [obuf0, obuf1]
        semi = [semi0, semi1]
        semo = [semo0, semo1]

        pltpu.sync_copy(part_hbm, pbuf)
        accs = []
        for a in range(NACC):
            v = jnp.zeros((L,), jnp.float32)
            for w in range(NW):
                v = v + pbuf[w, 0, pl.ds(a * L, L)]
            accs.append(jnp.sum(v))
        t0, t1, t2, c1, c2 = accs
        n_elems = jnp.float32(N_PIX * 3)
        s2 = (t2 - t1) * 0.5
        s1 = t1 - 2.0 * s2
        s0 = t0 - s1 - s2
        n2 = 3.0 * (c2 - c1) * 0.5
        n1 = 3.0 * c1 - 2.0 * n2
        n0 = n_elems - n1 - n2
        m0 = jnp.full((L,), s0, jnp.float32) / jnp.full((L,), n0, jnp.float32)
        m1 = jnp.full((L,), s1, jnp.float32) / jnp.full((L,), n1, jnp.float32)
        m2 = jnp.full((L,), s2, jnp.float32) / jnp.full((L,), n2, jnp.float32)

        def start_in(g, slot):
            return pltpu.async_copy(
                inst_hbm.at[pl.ds(irow0 + g * IROWS, IROWS), :],
                ibuf[slot], semi[slot])

        pend_in = [None, None]
        pend_out = [None, None]
        pend_in[0] = start_in(0, 0)
        for g in range(N_CHUNK):
            slot = g % 2
            if g + 1 < N_CHUNK:
                pend_in[(g + 1) % 2] = start_in(g + 1, (g + 1) % 2)
            pend_in[slot].wait()
            if pend_out[slot] is not None:
                for hdl in pend_out[slot]:
                    hdl.wait()
            islot = ibuf[slot]
            oslot = obuf[slot]

            def body(i, _, islot=islot, oslot=oslot):
                hr = i >> 5
                wc = pl.multiple_of((i & 31) << 4, 16)
                ir = i >> 3
                ic = pl.multiple_of((i & 7) << 4, 16)
                ci = islot[ir, pl.ds(ic, L)]
                v = jnp.where(ci == 0, m0, jnp.where(ci == 1, m1, m2))
                oslot[hr, pl.ds(wc, L)] = v
                return 0

            lax.fori_loop(0, GROUPS, body, 0, unroll=4)
            hs = []
            for c in range(C):
                r = fbase + c * H + g * CH_H
                hs.append(pltpu.async_copy(
                    oslot, out_hbm.at[pl.ds(r, CH_H), :], semo[slot]))
            pend_out[slot] = hs
        for slot in range(2):
            if pend_out[slot] is not None:
                for hdl in pend_out[slot]:
                    hdl.wait()

    def run(feats, inst):
        ft = feats.transpose(0, 3, 1, 2).reshape(B * C * H, W)
        ii = inst.reshape(N_PIX // 128, 128)
        part = phase1(ft, ii)
        out2d = phase2(ii, part)
        return out2d.reshape(B, C, H, W).transpose(0, 2, 3, 1)

    return run


_make_kernels_cached = functools.lru_cache(maxsize=None)(_make_kernels)


@jax.jit
def kernel(feats, inst):
    return _make_kernels_cached()(feats, inst)
